# Initial kernel scaffold; baseline (speedup 1.0000x reference)
import jax
import jax.numpy as jnp
from jax.experimental import pallas as pl


def kernel(adj, adj_bias_w, vt_bias_w):
    raise NotImplementedError("write your pallas kernel here")


# speedup vs baseline: 6.2316x; 6.2316x over previous
import numpy as np
import jax
import jax.numpy as jnp
from jax import lax
from jax.experimental import pallas as pl
from jax.experimental.pallas import tpu as pltpu
from jax.experimental.pallas import tpu_sc as plsc

B, N, H = 64, 128, 32
NP1 = N + 1
P = NP1 * NP1
L = 16
NCHUNK = (P + L - 1) // L
PPAD = NCHUNK * L
NC, NS = 2, 16
NW = NC * NS
GPW = B // NW
NSTAGE = N * N + 8

_m = np.arange(PPAD, dtype=np.int64)
_i, _j = _m // NP1, _m % NP1
_interior = (_m < P) & (_i >= 1) & (_j >= 1)
_border = (_m < P) & ~_interior
_SRCIDX = np.where(_interior, (_i - 1) * N + (_j - 1),
                   np.where(_border, N * N, N * N + 1)).astype(np.int32)
_MARKER = np.array([2, 0, 0, 0, 0, 0, 0, 0], dtype=np.int32)


def _sc_body(adj_hbm, wv_hbm, vt_hbm, srcidx_hbm, marker_hbm, out_hbm,
             staging, srcidx_v, tmpl, buf0, buf1, wv_v, vt_v, sem0, sem1):
    wid = lax.axis_index("s") * NC + lax.axis_index("c")

    pltpu.sync_copy(wv_hbm, wv_v)
    pltpu.sync_copy(vt_hbm, vt_v)
    pltpu.sync_copy(srcidx_hbm, srcidx_v)
    pltpu.sync_copy(marker_hbm, staging.at[pl.ds(N * N, 8)])

    bufs = (buf0, buf1)
    sems = (sem0, sem1)
    handles = [None, None]
    step = 0

    for g in range(GPW):
        b = wid * GPW + g
        pltpu.sync_copy(adj_hbm.at[b], staging.at[pl.ds(0, N * N)])

        def build(k, c):
            idx = srcidx_v[pl.ds(k * L, L)]
            t = plsc.load_gather(staging, [idx])
            tmpl[pl.ds(k * L, L)] = t.astype(jnp.float32)
            return c

        lax.fori_loop(0, NCHUNK, build, 0)

        for h in range(H):
            p = step % 2
            if handles[p] is not None:
                handles[p].wait()
            wv = wv_v[h]
            vt = vt_v[h]
            buf = bufs[p]

            def fill(k, c, buf=buf, wv=wv, vt=vt):
                t = tmpl[pl.ds(k * L, L)]
                buf[pl.ds(k * L, L)] = jnp.where(t > 1.5, vt, t * wv)
                return c

            lax.fori_loop(0, NCHUNK, fill, 0)
            handles[p] = pltpu.async_copy(
                buf.at[pl.ds(0, P)], out_hbm.at[b, h], sems[p])
            step += 1

    for p in range(2):
        if handles[p] is not None:
            handles[p].wait()


def kernel(adj, adj_bias_w, vt_bias_w):
    adj2 = adj.reshape(B, N * N)
    wv = jnp.broadcast_to(adj_bias_w[1][:, None], (H, L))
    vt = jnp.broadcast_to(vt_bias_w[0][:, None], (H, L))
    run = pl.kernel(
        _sc_body,
        out_type=jax.ShapeDtypeStruct((B, H, P), jnp.float32),
        mesh=plsc.VectorSubcoreMesh(core_axis_name="c", subcore_axis_name="s"),
        compiler_params=pltpu.CompilerParams(
            needs_layout_passes=False, use_tc_tiling_on_sc=False),
        scratch_types=[
            pltpu.VMEM((NSTAGE,), jnp.int32),
            pltpu.VMEM((PPAD,), jnp.int32),
            pltpu.VMEM((PPAD,), jnp.float32),
            pltpu.VMEM((PPAD,), jnp.float32),
            pltpu.VMEM((PPAD,), jnp.float32),
            pltpu.VMEM((H, L), jnp.float32),
            pltpu.VMEM((H, L), jnp.float32),
            pltpu.SemaphoreType.DMA,
            pltpu.SemaphoreType.DMA,
        ],
    )
    out = run(adj2, wv, vt, jnp.asarray(_SRCIDX), jnp.asarray(_MARKER))
    return out.reshape(B, H, NP1, NP1)


# speedup vs baseline: 8.6611x; 1.3899x over previous
import numpy as np
import jax
import jax.numpy as jnp
from jax import lax
from jax.experimental import pallas as pl
from jax.experimental.pallas import tpu as pltpu
from jax.experimental.pallas import tpu_sc as plsc

B, N, H = 64, 128, 32
NP1 = N + 1
P = NP1 * NP1
L = 16
NCHUNK = (P + L - 1) // L
PPAD = NCHUNK * L
NC, NS = 2, 16
NW = NC * NS
GPW = B // NW
NSTAGE = N * N + 8

_m = np.arange(PPAD, dtype=np.int64)
_i, _j = _m // NP1, _m % NP1
_interior = (_m < P) & (_i >= 1) & (_j >= 1)
_border = (_m < P) & ~_interior
_SRCIDX = np.where(_interior, (_i - 1) * N + (_j - 1),
                   np.where(_border, N * N, N * N + 1)).astype(np.int32)
_MARKER = np.array([2, 0, 0, 0, 0, 0, 0, 0], dtype=np.int32)


def _sc_body(adj_hbm, wv_hbm, vt_hbm, srcidx_hbm, marker_hbm, out_hbm,
             staging, srcidx_v, tmpl, buf0, buf1, wv_v, vt_v, sem0, sem1):
    wid = lax.axis_index("s") * NC + lax.axis_index("c")

    pltpu.sync_copy(wv_hbm, wv_v)
    pltpu.sync_copy(vt_hbm, vt_v)
    pltpu.sync_copy(srcidx_hbm, srcidx_v)
    pltpu.sync_copy(marker_hbm, staging.at[pl.ds(N * N, 8)])

    bufs = (buf0, buf1)
    sems = (sem0, sem1)
    handles = [None, None]
    step = 0

    for g in range(GPW):
        b = wid * GPW + g
        pltpu.sync_copy(adj_hbm.at[b], staging.at[pl.ds(0, N * N)])

        @plsc.parallel_loop(0, NCHUNK, step=1, unroll=8)
        def build(k):
            idx = srcidx_v[pl.ds(k * L, L)]
            t = plsc.load_gather(staging, [idx])
            tmpl[pl.ds(k * L, L)] = t.astype(jnp.float32)

        for h in range(H):
            p = step % 2
            if handles[p] is not None:
                handles[p].wait()
            wv = wv_v[h]
            vt = vt_v[h]
            buf = bufs[p]

            @plsc.parallel_loop(0, NCHUNK, step=1, unroll=8)
            def fill(k, buf=buf, wv=wv, vt=vt):
                t = tmpl[pl.ds(k * L, L)]
                buf[pl.ds(k * L, L)] = jnp.where(t > 1.5, vt, t * wv)
            handles[p] = pltpu.async_copy(
                buf.at[pl.ds(0, P)], out_hbm.at[b, h], sems[p])
            step += 1

    for p in range(2):
        if handles[p] is not None:
            handles[p].wait()


def kernel(adj, adj_bias_w, vt_bias_w):
    adj2 = adj.reshape(B, N * N)
    wv = jnp.broadcast_to(adj_bias_w[1][:, None], (H, L))
    vt = jnp.broadcast_to(vt_bias_w[0][:, None], (H, L))
    run = pl.kernel(
        _sc_body,
        out_type=jax.ShapeDtypeStruct((B, H, P), jnp.float32),
        mesh=plsc.VectorSubcoreMesh(core_axis_name="c", subcore_axis_name="s"),
        compiler_params=pltpu.CompilerParams(
            needs_layout_passes=False, use_tc_tiling_on_sc=False),
        scratch_types=[
            pltpu.VMEM((NSTAGE,), jnp.int32),
            pltpu.VMEM((PPAD,), jnp.int32),
            pltpu.VMEM((PPAD,), jnp.float32),
            pltpu.VMEM((PPAD,), jnp.float32),
            pltpu.VMEM((PPAD,), jnp.float32),
            pltpu.VMEM((H, L), jnp.float32),
            pltpu.VMEM((H, L), jnp.float32),
            pltpu.SemaphoreType.DMA,
            pltpu.SemaphoreType.DMA,
        ],
    )
    out = run(adj2, wv, vt, jnp.asarray(_SRCIDX), jnp.asarray(_MARKER))
    return out.reshape(B, H, NP1, NP1)


# speedup vs baseline: 9.4438x; 1.0904x over previous
import numpy as np
import jax
import jax.numpy as jnp
from jax import lax
from jax.experimental import pallas as pl
from jax.experimental.pallas import tpu as pltpu
from jax.experimental.pallas import tpu_sc as plsc

B, N, H = 64, 128, 32
NP1 = N + 1
L = 16
CPR = NP1 // L + 1
W = CPR * L
NCH = NP1 * CPR
NFILL = NP1 * (N // L)
NC, NS = 2, 16
NW = NC * NS
GPW = B // NW

_i = np.arange(NP1, dtype=np.int64)[:, None]
_c = np.arange(W, dtype=np.int64)[None, :]
_interior = (_i >= 1) & (_c >= 1) & (_c <= N)
_border = ((_i == 0) & (_c <= N)) | (_c == 0)
_SRCIDX = np.where(_interior, (_i - 1) * N + (_c - 1),
                   np.where(_border, N * N, N * N + 1))
_SRCIDX = _SRCIDX.reshape(-1).astype(np.int32)
_MARKER = np.array([2, 0, 0, 0, 0, 0, 0, 0], dtype=np.int32)


def _sc_body(adj_hbm, wv_hbm, vt_hbm, srcidx_hbm, marker_hbm, out_hbm,
             staging, srcidx_v, tmpl, buf0, buf1, wv_v, vt_v, sem0, sem1):
    wid = lax.axis_index("s") * NC + lax.axis_index("c")

    pltpu.sync_copy(wv_hbm, wv_v)
    pltpu.sync_copy(vt_hbm, vt_v)
    pltpu.sync_copy(srcidx_hbm, srcidx_v)
    pltpu.sync_copy(marker_hbm, staging.at[pl.ds(N * N, 8)])

    bufs = (buf0, buf1)
    sems = (sem0, sem1)
    handles = [None, None]
    step = 0
    lanes = lax.iota(jnp.int32, L)
    col128 = jnp.full((L,), N, dtype=jnp.int32)

    for g in range(GPW):
        b = wid * GPW + g
        pltpu.sync_copy(adj_hbm.at[b], staging.at[pl.ds(0, N * N)])

        @plsc.parallel_loop(0, NCH, step=1, unroll=8)
        def build(k):
            idx = srcidx_v[pl.ds(k * L, L)]
            t = plsc.load_gather(staging, [idx])
            tmpl[pl.ds(k * L, L)] = t.astype(jnp.float32)

        for h in range(H):
            p = step % 2
            if handles[p] is not None:
                handles[p].wait()
            wv = wv_v[h]
            vt = vt_v[h]
            buf = bufs[p]

            @plsc.parallel_loop(0, NFILL, step=1, unroll=8)
            def fill(k, buf=buf, wv=wv, vt=vt):
                i = k // (N // L)
                c = k - i * (N // L)
                t = tmpl[pl.ds(i * W + c * L, L)]
                buf[i, pl.ds(c * L, L)] = jnp.where(t > 1.5, vt, t * wv)

            @plsc.parallel_loop(0, CPR, step=1)
            def fill_col(r, buf=buf, wv=wv, vt=vt):
                rows = lanes + r * L
                mask = rows <= N
                safe = jnp.where(mask, rows, 0)
                t = plsc.load_gather(tmpl, [safe * W + N])
                val = jnp.where(t > 1.5, vt, t * wv)
                plsc.store_scatter(buf, [safe, col128], val, mask=mask)

            handles[p] = pltpu.async_copy(buf, out_hbm.at[b, h], sems[p])
            step += 1

    for p in range(2):
        if handles[p] is not None:
            handles[p].wait()


def kernel(adj, adj_bias_w, vt_bias_w):
    adj2 = adj.reshape(B, N * N)
    wv = jnp.broadcast_to(adj_bias_w[1][:, None], (H, L))
    vt = jnp.broadcast_to(vt_bias_w[0][:, None], (H, L))
    run = pl.kernel(
        _sc_body,
        out_type=jax.ShapeDtypeStruct((B, H, NP1, NP1), jnp.float32),
        mesh=plsc.VectorSubcoreMesh(core_axis_name="c", subcore_axis_name="s"),
        compiler_params=pltpu.CompilerParams(
            needs_layout_passes=False, use_tc_tiling_on_sc=False),
        scratch_types=[
            pltpu.VMEM((N * N + 8,), jnp.int32),
            pltpu.VMEM((NP1 * W,), jnp.int32),
            pltpu.VMEM((NP1 * W,), jnp.float32),
            pltpu.VMEM((NP1, NP1), jnp.float32),
            pltpu.VMEM((NP1, NP1), jnp.float32),
            pltpu.VMEM((H, L), jnp.float32),
            pltpu.VMEM((H, L), jnp.float32),
            pltpu.SemaphoreType.DMA,
            pltpu.SemaphoreType.DMA,
        ],
    )
    return run(adj2, wv, vt, jnp.asarray(_SRCIDX), jnp.asarray(_MARKER))


# speedup vs baseline: 19.2613x; 2.0396x over previous
import numpy as np
import jax
import jax.numpy as jnp
from jax import lax
from jax.experimental import pallas as pl
from jax.experimental.pallas import tpu as pltpu
from jax.experimental.pallas import tpu_sc as plsc

B, N, H = 64, 128, 32
NP1 = N + 1
L = 16
CPL = N // L
NFILL = NP1 * CPL
CCH = NP1 // L + 1
NC, NS = 2, 16
NW = NC * NS
GPW = B // NW
NSTAGE = N * N + 128
NIDX = NFILL * L + CCH * L
NIDXP = -(-NIDX // 128) * 128

_k = np.arange(NFILL, dtype=np.int64)
_lane = np.arange(L, dtype=np.int64)
_i = (_k // CPL)[:, None] + 0 * _lane
_c = ((_k % CPL) * L)[:, None] + _lane
_interior = (_i >= 1) & (_c >= 1)
_aligned = np.where(_interior, (_i - 1) * N + (_c - 1), N * N)
_r = np.arange(CCH, dtype=np.int64)
_rows = (_r * L)[:, None] + _lane
_colsrc = np.where(_rows == 0, N * N,
                   np.where(_rows <= N, (_rows - 1) * N + (N - 1),
                            N * N + 1))
_SRCIDX = np.full((NIDXP,), N * N + 1, dtype=np.int32)
_SRCIDX[:NFILL * L] = _aligned.reshape(-1)
_SRCIDX[NFILL * L:NIDX] = _colsrc.reshape(-1)
_MARKER = np.zeros((128,), dtype=np.int32)
_MARKER[0] = 2


def _sc_body(adj_hbm, wv_hbm, vt_hbm, srcidx_hbm, marker_hbm, out_hbm,
             staging, srcidx_v, buf0, buf1, wv_v, vt_v, sem0, sem1):
    wid = lax.axis_index("s") * NC + lax.axis_index("c")

    pltpu.sync_copy(wv_hbm, wv_v)
    pltpu.sync_copy(vt_hbm, vt_v)
    pltpu.sync_copy(srcidx_hbm, srcidx_v)
    pltpu.sync_copy(marker_hbm, staging.at[pl.ds(N * N, 128)])

    bufs = (buf0, buf1)
    sems = (sem0, sem1)
    handles = [None, None]
    step = 0
    lanes = lax.iota(jnp.int32, L)
    col128 = jnp.full((L,), N, dtype=jnp.int32)

    for g in range(GPW):
        b = wid * GPW + g
        pltpu.sync_copy(adj_hbm.at[b], staging.at[pl.ds(0, N * N)])

        for h in range(H):
            p = step % 2
            if handles[p] is not None:
                handles[p].wait()
            wv = wv_v[pl.ds(h * L, L)]
            vt = vt_v[pl.ds(h * L, L)]
            buf = bufs[p]

            @plsc.parallel_loop(0, NFILL, step=1, unroll=8)
            def fill(k, buf=buf, wv=wv, vt=vt):
                i = k // CPL
                c = k - i * CPL
                idx = srcidx_v[pl.ds(k * L, L)]
                t = plsc.load_gather(staging, [idx]).astype(jnp.float32)
                buf[i, pl.ds(c * L, L)] = jnp.where(t > 1.5, vt, t * wv)

            @plsc.parallel_loop(0, CCH, step=1)
            def fill_col(r, buf=buf, wv=wv, vt=vt):
                idx = srcidx_v[pl.ds(NFILL * L + r * L, L)]
                t = plsc.load_gather(staging, [idx]).astype(jnp.float32)
                val = jnp.where(t > 1.5, vt, t * wv)
                rows = lanes + r * L
                mask = rows <= N
                safe = jnp.where(mask, rows, 0)
                plsc.store_scatter(buf, [safe, col128], val, mask=mask)

            handles[p] = pltpu.async_copy(buf, out_hbm.at[b, h], sems[p])
            step += 1

    for p in range(2):
        if handles[p] is not None:
            handles[p].wait()


def kernel(adj, adj_bias_w, vt_bias_w):
    adj2 = adj.reshape(B, N * N)
    wv = jnp.broadcast_to(adj_bias_w[1][:, None], (H, L)).reshape(H * L)
    vt = jnp.broadcast_to(vt_bias_w[0][:, None], (H, L)).reshape(H * L)
    run = pl.kernel(
        _sc_body,
        out_type=jax.ShapeDtypeStruct((B, H, NP1, NP1), jnp.float32),
        mesh=plsc.VectorSubcoreMesh(core_axis_name="c", subcore_axis_name="s"),
        compiler_params=pltpu.CompilerParams(
            needs_layout_passes=False, use_tc_tiling_on_sc=True),
        scratch_types=[
            pltpu.VMEM((NSTAGE,), jnp.int32),
            pltpu.VMEM((NIDXP,), jnp.int32),
            pltpu.VMEM((NP1, NP1), jnp.float32),
            pltpu.VMEM((NP1, NP1), jnp.float32),
            pltpu.VMEM((H * L,), jnp.float32),
            pltpu.VMEM((H * L,), jnp.float32),
            pltpu.SemaphoreType.DMA,
            pltpu.SemaphoreType.DMA,
        ],
    )
    return run(adj2, wv, vt, jnp.asarray(_SRCIDX), jnp.asarray(_MARKER))


# speedup vs baseline: 48.4935x; 2.5177x over previous
import numpy as np
import jax
import jax.numpy as jnp
from jax import lax
from jax.experimental import pallas as pl
from jax.experimental.pallas import tpu as pltpu
from jax.experimental.pallas import tpu_sc as plsc

B, N, H = 64, 128, 32
NP1 = N + 1
L = 16
CPL = N // L
CPR = CPL + 1
W = CPR * L
NTCH = NP1 * CPR
NC, NS = 2, 16
NW = NC * NS
GPW = B // NW
NSTAGE = N * N + 128
NI = 4
NGRP = NP1 // NI
TAIL = NP1 - NI * NGRP
NPAIR = (NGRP - 2) // 2
HCH = H // L

_MARKER = np.zeros((128,), dtype=np.int32)
_MARKER[0] = 2


def _sc_body(adj_hbm, wv_hbm, vt_hbm, marker_hbm, out_hbm,
             staging, tmpl, buf0, buf1, wv_v, vt_v, sem0, sem1):
    wid = lax.axis_index("s") * NC + lax.axis_index("c")

    pltpu.sync_copy(wv_hbm, wv_v)
    pltpu.sync_copy(vt_hbm, vt_v)
    pltpu.sync_copy(marker_hbm, staging.at[pl.ds(N * N, 128)])

    lanes = lax.iota(jnp.int32, L)
    colN = jnp.full((L,), N, dtype=jnp.int32)

    def fill_slab(i0, ni, buf):
        @plsc.parallel_loop(0, ni * H, step=1)
        def fill(k, i0=i0, buf=buf):
            r = k // H
            h = k - r * H
            hv = jnp.full((L,), h, dtype=jnp.int32)
            wv = plsc.load_gather(wv_v, [hv])
            vt = plsc.load_gather(vt_v, [hv])
            base = (i0 + r) * W
            for c in range(CPL):
                t = tmpl[pl.ds(base + c * L, L)]
                buf[r, h, pl.ds(c * L, L)] = jnp.where(t > 1.5, vt, t * wv)

        @plsc.parallel_loop(0, ni * HCH, step=1)
        def fill_col(k, i0=i0, buf=buf):
            r = k // HCH
            hc = k - r * HCH
            tv = plsc.load_gather(
                tmpl, [jnp.full((L,), (i0 + r) * W + N, dtype=jnp.int32)])
            wvc = wv_v[pl.ds(hc * L, L)]
            vtc = vt_v[pl.ds(hc * L, L)]
            val = jnp.where(tv > 1.5, vtc, tv * wvc)
            plsc.store_scatter(buf.at[r], [lanes + hc * L, colN], val)

    def start(buf, b, i0, ni, sem):
        return pltpu.async_copy(
            buf.at[pl.ds(0, ni)], out_hbm.at[b, pl.ds(i0, ni)], sem)

    def wait(buf, ni, sem):
        pltpu.make_async_copy(
            buf.at[pl.ds(0, ni)], out_hbm.at[0, pl.ds(0, ni)], sem).wait()

    bufs = (buf0, buf1)
    sems = (sem0, sem1)
    pend = [[], []]

    for g in range(GPW):
        b = wid * GPW + g
        pltpu.sync_copy(adj_hbm.at[b], staging.at[pl.ds(0, N * N)])

        @plsc.parallel_loop(0, NTCH, step=1, unroll=8)
        def build(k):
            i = k // CPR
            c = k - i * CPR
            col = lanes + c * L
            src = (i - 1) * N + col - 1
            idx = jnp.where((col == 0) | (i == 0), N * N,
                            jnp.where(col > N, N * N + 1, src))
            t = plsc.load_gather(staging, [idx])
            tmpl[pl.ds(k * L, L)] = t.astype(jnp.float32)

        for q in (0, 1):
            if pend[q]:
                wait(bufs[q], pend[q].pop(), sems[q])
            fill_slab(q * NI, NI, bufs[q])
            start(bufs[q], b, q * NI, NI, sems[q])

        def body(j, carry):
            i0 = (2 * j + 2) * NI
            wait(buf0, NI, sem0)
            fill_slab(i0, NI, buf0)
            start(buf0, b, i0, NI, sem0)
            wait(buf1, NI, sem1)
            fill_slab(i0 + NI, NI, buf1)
            start(buf1, b, i0 + NI, NI, sem1)
            return carry

        lax.fori_loop(0, NPAIR, body, 0)

        wait(buf0, NI, sem0)
        fill_slab(NGRP * NI, TAIL, buf0)
        start(buf0, b, NGRP * NI, TAIL, sem0)
        pend[0] = [TAIL]
        pend[1] = [NI]

    wait(buf0, pend[0].pop(), sem0)
    wait(buf1, pend[1].pop(), sem1)


def kernel(adj, adj_bias_w, vt_bias_w):
    adj2 = adj.reshape(B, N * N)
    run = pl.kernel(
        _sc_body,
        out_type=jax.ShapeDtypeStruct((B, NP1, H, NP1), jnp.float32),
        mesh=plsc.VectorSubcoreMesh(core_axis_name="c", subcore_axis_name="s"),
        compiler_params=pltpu.CompilerParams(
            needs_layout_passes=False, use_tc_tiling_on_sc=True),
        scratch_types=[
            pltpu.VMEM((NSTAGE,), jnp.int32),
            pltpu.VMEM((NP1 * W,), jnp.float32),
            pltpu.VMEM((NI, H, NP1), jnp.float32),
            pltpu.VMEM((NI, H, NP1), jnp.float32),
            pltpu.VMEM((128,), jnp.float32),
            pltpu.VMEM((128,), jnp.float32),
            pltpu.SemaphoreType.DMA,
            pltpu.SemaphoreType.DMA,
        ],
    )
    wv = jnp.zeros((128,), jnp.float32).at[:H].set(adj_bias_w[1])
    vt = jnp.zeros((128,), jnp.float32).at[:H].set(vt_bias_w[0])
    out = run(adj2, wv, vt, jnp.asarray(_MARKER))
    return out.transpose(0, 2, 1, 3)
